# baseline (speedup 1.0000x reference)
import functools

import jax
import jax.numpy as jnp
from jax.experimental import pallas as pl
from jax.experimental.pallas import tpu as pltpu

_NUM_CLUSTERS = 64
_N_LEVELS = 127.0


def _rowabs_kernel(x_ref, rabs_ref):
    rabs_ref[...] = jnp.max(jnp.abs(x_ref[...]), axis=1, keepdims=True)


def _segscale_kernel(rabs_ref, c_ref, scale_ref):
    r = rabs_ref[...]
    c = c_ref[...]

    lane = jax.lax.broadcasted_iota(jnp.int32, (1, _NUM_CLUSTERS), 1)

    def body(k, acc):
        m = jnp.max(jnp.where(c == k, r, 0.0))
        return jnp.where(lane == k, m, acc)

    sat = jax.lax.fori_loop(
        0, _NUM_CLUSTERS, body, jnp.zeros((1, _NUM_CLUSTERS), jnp.float32)
    )
    scale_ref[...] = jnp.maximum(sat, 1e-8) / _N_LEVELS


def _quant_kernel(x_ref, c_ref, scale_ref, out_ref):
    x = x_ref[...]
    c = c_ref[...]
    scale = scale_ref[...]
    ids = jax.lax.broadcasted_iota(jnp.int32, (x.shape[0], _NUM_CLUSTERS), 1)
    mask = c == ids
    rs = jnp.sum(jnp.where(mask, scale, 0.0), axis=1, keepdims=True)
    q = jnp.clip(jnp.round(x / rs), -_N_LEVELS - 1.0, _N_LEVELS)
    out_ref[...] = q * rs


@functools.partial(jax.jit, static_argnames=())
def kernel(x, cluster):
    n, d = x.shape
    block_rows = 16000
    nb = n // block_rows

    c2d = cluster.reshape(n, 1).astype(jnp.int32)

    rabs = pl.pallas_call(
        _rowabs_kernel,
        grid=(nb,),
        in_specs=[pl.BlockSpec((block_rows, d), lambda i: (i, 0))],
        out_specs=pl.BlockSpec((block_rows, 1), lambda i: (i, 0)),
        out_shape=jax.ShapeDtypeStruct((n, 1), jnp.float32),
        compiler_params=pltpu.CompilerParams(
            dimension_semantics=("arbitrary",),
        ),
    )(x)

    rabs_dense = rabs.reshape(n // d, d)
    c_dense = cluster.reshape(n // d, d).astype(jnp.int32)

    scale = pl.pallas_call(
        _segscale_kernel,
        out_shape=jax.ShapeDtypeStruct((1, _NUM_CLUSTERS), jnp.float32),
    )(rabs_dense, c_dense)

    out = pl.pallas_call(
        _quant_kernel,
        grid=(nb,),
        in_specs=[
            pl.BlockSpec((block_rows, d), lambda i: (i, 0)),
            pl.BlockSpec((block_rows, 1), lambda i: (i, 0)),
            pl.BlockSpec((1, _NUM_CLUSTERS), lambda i: (0, 0)),
        ],
        out_specs=pl.BlockSpec((block_rows, d), lambda i: (i, 0)),
        out_shape=jax.ShapeDtypeStruct((n, d), jnp.float32),
        compiler_params=pltpu.CompilerParams(
            dimension_semantics=("arbitrary",),
        ),
    )(x, c2d, scale)

    return out, scale.reshape(_NUM_CLUSTERS)


# speedup vs baseline: 1.2104x; 1.2104x over previous
import functools

import jax
import jax.numpy as jnp
from jax.experimental import pallas as pl
from jax.experimental.pallas import tpu as pltpu

_NUM_CLUSTERS = 64
_N_LEVELS = 127.0


def _stats_kernel(x_ref, c_ref, pabs_ref):
    x = x_ref[...]
    c = c_ref[...]
    rabs = jnp.max(jnp.abs(x), axis=1, keepdims=True)
    ids = jax.lax.broadcasted_iota(jnp.int32, (x.shape[0], _NUM_CLUSTERS), 1)
    mask = c == ids
    pabs_ref[...] = jnp.max(jnp.where(mask, rabs, 0.0), axis=0, keepdims=True)[None]


def _scale_kernel(pabs_ref, scale_ref):
    sat = jnp.max(pabs_ref[...], axis=(0, 1))[None]
    scale = jnp.maximum(sat, 1e-8) / _N_LEVELS
    scale_ref[...] = jnp.concatenate([scale, 1.0 / scale], axis=0)


def _quant_kernel(x_ref, c_ref, scale_ref, out_ref):
    x = x_ref[...]
    c = c_ref[...]
    scale = scale_ref[0:1, :]
    inv = scale_ref[1:2, :]
    ids = jax.lax.broadcasted_iota(jnp.int32, (x.shape[0], _NUM_CLUSTERS), 1)
    mask = c == ids
    rs = jnp.sum(jnp.where(mask, scale, 0.0), axis=1, keepdims=True)
    ri = jnp.sum(jnp.where(mask, inv, 0.0), axis=1, keepdims=True)
    q = jnp.clip(jnp.round(x * ri), -_N_LEVELS - 1.0, _N_LEVELS)
    out_ref[...] = q * rs


@functools.partial(jax.jit, static_argnames=())
def kernel(x, cluster):
    n, d = x.shape
    block_rows = 16000
    nb = n // block_rows

    c2d = cluster.reshape(n, 1).astype(jnp.int32)

    pabs = pl.pallas_call(
        _stats_kernel,
        grid=(nb,),
        in_specs=[
            pl.BlockSpec((block_rows, d), lambda i: (i, 0)),
            pl.BlockSpec((block_rows, 1), lambda i: (i, 0)),
        ],
        out_specs=pl.BlockSpec((1, 1, _NUM_CLUSTERS), lambda i: (i, 0, 0)),
        out_shape=jax.ShapeDtypeStruct((nb, 1, _NUM_CLUSTERS), jnp.float32),
        compiler_params=pltpu.CompilerParams(
            dimension_semantics=("arbitrary",),
        ),
    )(x, c2d)

    scale = pl.pallas_call(
        _scale_kernel,
        out_shape=jax.ShapeDtypeStruct((2, _NUM_CLUSTERS), jnp.float32),
    )(pabs)

    q_rows = 8000
    nq = n // q_rows
    out = pl.pallas_call(
        _quant_kernel,
        grid=(nq,),
        in_specs=[
            pl.BlockSpec((q_rows, d), lambda i: (i, 0)),
            pl.BlockSpec((q_rows, 1), lambda i: (i, 0)),
            pl.BlockSpec((2, _NUM_CLUSTERS), lambda i: (0, 0)),
        ],
        out_specs=pl.BlockSpec((q_rows, d), lambda i: (i, 0)),
        out_shape=jax.ShapeDtypeStruct((n, d), jnp.float32),
        compiler_params=pltpu.CompilerParams(
            dimension_semantics=("arbitrary",),
        ),
    )(x, c2d, scale)

    return out, scale[0].reshape(_NUM_CLUSTERS)


# speedup vs baseline: 1.2517x; 1.0341x over previous
import functools

import jax
import jax.numpy as jnp
from jax.experimental import pallas as pl
from jax.experimental.pallas import tpu as pltpu

_NUM_CLUSTERS = 64
_N_LEVELS = 127.0


def _stats_kernel(x_ref, c_ref, pabs_ref):
    x = x_ref[...]
    c = c_ref[...]
    rabs = jnp.max(jnp.abs(x), axis=1, keepdims=True)
    ids = jax.lax.broadcasted_iota(jnp.int32, (x.shape[0], _NUM_CLUSTERS), 1)
    mask = c == ids
    pabs_ref[...] = jnp.max(jnp.where(mask, rabs, 0.0), axis=0, keepdims=True)[None]


def _scale_kernel(pabs_ref, scale_ref):
    sat = jnp.max(pabs_ref[...], axis=(0, 1))[None]
    scale_ref[...] = jnp.maximum(sat, 1e-8) / _N_LEVELS


def _quant_kernel(x_ref, c_ref, scale_ref, out_ref):
    x = x_ref[...]
    c = c_ref[...]
    scale = scale_ref[...]
    ids = jax.lax.broadcasted_iota(jnp.int32, (x.shape[0], _NUM_CLUSTERS), 1)
    mask = c == ids
    rs = jnp.sum(jnp.where(mask, scale, 0.0), axis=1, keepdims=True)
    q = jnp.clip(jnp.round(x / rs), -_N_LEVELS - 1.0, _N_LEVELS)
    out_ref[...] = q * rs


@functools.partial(jax.jit, static_argnames=())
def kernel(x, cluster):
    n, d = x.shape
    block_rows = 16000
    nb = n // block_rows

    c2d = cluster.reshape(n, 1).astype(jnp.int32)

    pabs = pl.pallas_call(
        _stats_kernel,
        grid=(nb,),
        in_specs=[
            pl.BlockSpec((block_rows, d), lambda i: (i, 0)),
            pl.BlockSpec((block_rows, 1), lambda i: (i, 0)),
        ],
        out_specs=pl.BlockSpec((1, 1, _NUM_CLUSTERS), lambda i: (i, 0, 0)),
        out_shape=jax.ShapeDtypeStruct((nb, 1, _NUM_CLUSTERS), jnp.float32),
        compiler_params=pltpu.CompilerParams(
            dimension_semantics=("arbitrary",),
        ),
    )(x, c2d)

    scale = pl.pallas_call(
        _scale_kernel,
        out_shape=jax.ShapeDtypeStruct((1, _NUM_CLUSTERS), jnp.float32),
    )(pabs)

    out = pl.pallas_call(
        _quant_kernel,
        grid=(nb,),
        in_specs=[
            pl.BlockSpec((block_rows, d), lambda i: (i, 0)),
            pl.BlockSpec((block_rows, 1), lambda i: (i, 0)),
            pl.BlockSpec((1, _NUM_CLUSTERS), lambda i: (0, 0)),
        ],
        out_specs=pl.BlockSpec((block_rows, d), lambda i: (i, 0)),
        out_shape=jax.ShapeDtypeStruct((n, d), jnp.float32),
        compiler_params=pltpu.CompilerParams(
            dimension_semantics=("arbitrary",),
        ),
    )(x, c2d, scale)

    return out, scale.reshape(_NUM_CLUSTERS)
